# baseline (speedup 1.0000x reference)
import functools

import jax
import jax.numpy as jnp
from jax import lax
from jax.experimental import pallas as pl
from jax.experimental.pallas import tpu as pltpu
from jax.experimental.pallas import tpu_sc as plsc

_N = 10000
_E = 320000
_F = 128
_H = 32
_G = 128
_C = 10

_NC = 2
_NS = 16
_NW = _NC * _NS
_CH = 128
_CPW = 80
_TCH = _NW * _CPW
_IST = 40
_EPAD = _TCH * _CH
_NA = _N + 8
_RPT = 624
_REM = _N - _NS * _RPT


def _sc_agg_body(h_hbm, z_hbm, src_hbm, dst_hbm, out_hbm,
                 si0, di0, rows0, acc, sg0):
    cid = lax.axis_index("c")
    sid = lax.axis_index("s")
    wid = cid * _NS + sid
    r0 = sid * _RPT
    pltpu.sync_copy(z_hbm.at[pl.ds(r0, _RPT)], acc.at[pl.ds(r0, _RPT)])

    @pl.when(sid == _NS - 1)
    def _():
        pltpu.sync_copy(z_hbm.at[pl.ds(_NS * _RPT, _NA - _NS * _RPT)],
                        acc.at[pl.ds(_NS * _RPT, _NA - _NS * _RPT)])

    c0 = wid * _CPW
    plsc.subcore_barrier()

    @pl.loop(0, _CPW)
    def _(j):
        pltpu.sync_copy(src_hbm.at[c0 + j], si0)
        pltpu.sync_copy(dst_hbm.at[c0 + j], di0)
        pltpu.async_copy(h_hbm.at[si0], rows0, sg0).wait()
        pltpu.sync_copy(rows0, acc.at[di0], add=True)

    plsc.subcore_barrier()
    pltpu.sync_copy(acc.at[pl.ds(r0, _RPT)],
                    out_hbm.at[cid, pl.ds(r0, _RPT)])

    @pl.when(sid == _NS - 1)
    def _():
        pltpu.sync_copy(acc.at[pl.ds(_NS * _RPT, _REM)],
                        out_hbm.at[cid, pl.ds(_NS * _RPT, _REM)])


def _sc_agg(h, zeros, src, dst, width):
    mesh = plsc.VectorSubcoreMesh(core_axis_name="c", subcore_axis_name="s",
                                  num_cores=_NC, num_subcores=_NS)
    f = pl.kernel(
        _sc_agg_body,
        out_type=jax.ShapeDtypeStruct((_NC, _N, width), jnp.float32),
        mesh=mesh,
        scratch_types=[
            pltpu.VMEM((_CH,), jnp.int32),
            pltpu.VMEM((_CH,), jnp.int32),
            pltpu.VMEM((_CH, width), jnp.float32),
            pltpu.VMEM_SHARED((_NA, width), jnp.float32),
            pltpu.SemaphoreType.DMA,
        ],
        compiler_params=pltpu.CompilerParams(use_tc_tiling_on_sc=False),
    )
    return f(h, zeros, src, dst)


def _dot(a, b, prec=jax.lax.Precision.DEFAULT):
    return jnp.dot(a, b, preferred_element_type=jnp.float32, precision=prec)


def _mlp_bn(h, p0, p1, w1, b1, w2, b2, g, be):
    m = h + (p0 + p1)
    t = jnp.maximum(_dot(m, w1) + b1, 0.0)
    t = _dot(t, w2) + b2
    t = jnp.maximum(t, 0.0)
    mu = jnp.mean(t, axis=0, keepdims=True)
    var = jnp.mean((t - mu) * (t - mu), axis=0, keepdims=True)
    return g * (t - mu) / jnp.sqrt(var + 1e-5) + be


def _layer_body(h_ref, p_ref, w1_ref, b1_ref, w2_ref, b2_ref, g_ref, be_ref,
                o_ref):
    o_ref[...] = _mlp_bn(h_ref[...], p_ref[0], p_ref[1], w1_ref[...],
                         b1_ref[...], w2_ref[...], b2_ref[...], g_ref[...],
                         be_ref[...])


def _final_body(h_ref, p_ref, w1_ref, b1_ref, w2_ref, b2_ref, g_ref, be_ref,
                batch_ref, fc1w_ref, fc1b_ref, fc2w_ref, fc2b_ref, o_ref):
    h = _mlp_bn(h_ref[...], p_ref[0], p_ref[1], w1_ref[...], b1_ref[...],
                w2_ref[...], b2_ref[...], g_ref[...], be_ref[...])
    seg = batch_ref[...]
    onehot = (seg == lax.broadcasted_iota(jnp.int32, (_N, _G), 1)
              ).astype(jnp.float32)
    pooled = lax.dot_general(onehot, h, (((0,), (0,)), ((), ())),
                             preferred_element_type=jnp.float32,
                             precision=jax.lax.Precision.HIGHEST)
    z = jnp.maximum(_dot(pooled, fc1w_ref[...]) + fc1b_ref[...], 0.0)
    logits = _dot(z, fc2w_ref[...]) + fc2b_ref[...]
    mx = jnp.max(logits, axis=-1, keepdims=True)
    s = logits - mx
    o_ref[...] = s - jnp.log(jnp.sum(jnp.exp(s), axis=-1, keepdims=True))


def _tc(body, out_shape, *args):
    return pl.pallas_call(
        body, out_shape=jax.ShapeDtypeStruct(out_shape, jnp.float32))(*args)


def kernel(x, edge_index, batch,
           W1_0, b1_0, W2_0, b2_0, gamma_0, beta_0,
           W1_1, b1_1, W2_1, b2_1, gamma_1, beta_1,
           W1_2, b1_2, W2_2, b2_2, gamma_2, beta_2,
           fc1_W, fc1_b, fc2_W, fc2_b):
    npad = _EPAD - _E
    src = jnp.concatenate(
        [edge_index[0], jnp.zeros((npad,), jnp.int32)]).reshape(_TCH, _CH)
    dst = jnp.concatenate(
        [edge_index[1], jnp.full((npad,), _N, jnp.int32)]).reshape(_TCH, _CH)
    zeros_f = jnp.zeros((_NA, _F), jnp.float32)
    zeros_h = jnp.zeros((_NA, _H), jnp.float32)
    batch2d = batch.reshape(_N, 1)
    r2 = lambda v: v.reshape(1, -1)

    p = _sc_agg(x, zeros_f, src, dst, _F)
    h = _tc(_layer_body, (_N, _H), x, p, W1_0, r2(b1_0), W2_0, r2(b2_0),
            r2(gamma_0), r2(beta_0))
    p = _sc_agg(h, zeros_h, src, dst, _H)
    h = _tc(_layer_body, (_N, _H), h, p, W1_1, r2(b1_1), W2_1, r2(b2_1),
            r2(gamma_1), r2(beta_1))
    p = _sc_agg(h, zeros_h, src, dst, _H)
    return _tc(_final_body, (_G, _C), h, p, W1_2, r2(b1_2), W2_2, r2(b2_2),
               r2(gamma_2), r2(beta_2), batch2d, fc1_W, r2(fc1_b),
               fc2_W, r2(fc2_b))


# speedup vs baseline: 1.7779x; 1.7779x over previous
import functools

import jax
import jax.numpy as jnp
from jax import lax
from jax.experimental import pallas as pl
from jax.experimental.pallas import tpu as pltpu
from jax.experimental.pallas import tpu_sc as plsc

_N = 10000
_E = 320000
_F = 128
_H = 32
_G = 128
_C = 10

_NC = 2
_NS = 16
_NW = _NC * _NS
_EPW = _E // _NW
_CH = 128
_NFULL = _EPW // _CH
_TAIL = _EPW - _NFULL * _CH
_RPT = 624
_REM = _N - _NS * _RPT


def _sc_agg_body(h_hbm, z_hbm, src_hbm, dst_hbm, out_hbm,
                 si, di, rows, sit, dit, rowst, acc, sem):
    cid = lax.axis_index("c")
    sid = lax.axis_index("s")
    wid = cid * _NS + sid
    r0 = sid * _RPT
    pltpu.sync_copy(z_hbm.at[pl.ds(r0, _RPT)], acc.at[pl.ds(r0, _RPT)])

    @pl.when(sid == _NS - 1)
    def _():
        pltpu.sync_copy(z_hbm.at[pl.ds(_NS * _RPT, _REM)],
                        acc.at[pl.ds(_NS * _RPT, _REM)])

    plsc.subcore_barrier()
    base = wid * _EPW

    @pl.loop(0, _NFULL)
    def _(j):
        off = base + j * _CH
        pltpu.sync_copy(src_hbm.at[pl.ds(off, _CH)], si)
        pltpu.sync_copy(dst_hbm.at[pl.ds(off, _CH)], di)
        pltpu.async_copy(h_hbm.at[si], rows, sem).wait()
        pltpu.sync_copy(rows, acc.at[di], add=True)

    offt = base + _NFULL * _CH
    pltpu.sync_copy(src_hbm.at[pl.ds(offt, _TAIL)], sit)
    pltpu.sync_copy(dst_hbm.at[pl.ds(offt, _TAIL)], dit)
    pltpu.async_copy(h_hbm.at[sit], rowst, sem).wait()
    pltpu.sync_copy(rowst, acc.at[dit], add=True)

    plsc.subcore_barrier()
    pltpu.sync_copy(acc.at[pl.ds(r0, _RPT)],
                    out_hbm.at[cid, pl.ds(r0, _RPT)])

    @pl.when(sid == _NS - 1)
    def _():
        pltpu.sync_copy(acc.at[pl.ds(_NS * _RPT, _REM)],
                        out_hbm.at[cid, pl.ds(_NS * _RPT, _REM)])


def _sc_agg(h, zeros, src, dst, width):
    mesh = plsc.VectorSubcoreMesh(core_axis_name="c", subcore_axis_name="s",
                                  num_cores=_NC, num_subcores=_NS)
    f = pl.kernel(
        _sc_agg_body,
        out_type=jax.ShapeDtypeStruct((_NC, _N, width), jnp.float32),
        mesh=mesh,
        scratch_types=[
            pltpu.VMEM((_CH,), jnp.int32),
            pltpu.VMEM((_CH,), jnp.int32),
            pltpu.VMEM((_CH, width), jnp.float32),
            pltpu.VMEM((_TAIL,), jnp.int32),
            pltpu.VMEM((_TAIL,), jnp.int32),
            pltpu.VMEM((_TAIL, width), jnp.float32),
            pltpu.VMEM_SHARED((_N, width), jnp.float32),
            pltpu.SemaphoreType.DMA,
        ],
        compiler_params=pltpu.CompilerParams(use_tc_tiling_on_sc=False),
    )
    return f(h, zeros, src, dst)


def _dot(a, b, prec=jax.lax.Precision.DEFAULT):
    return jnp.dot(a, b, preferred_element_type=jnp.float32, precision=prec)


def _mlp_bn(h, p0, p1, w1, b1, w2, b2, g, be):
    m = h + (p0 + p1)
    t = jnp.maximum(_dot(m, w1) + b1, 0.0)
    t = _dot(t, w2) + b2
    t = jnp.maximum(t, 0.0)
    mu = jnp.mean(t, axis=0, keepdims=True)
    var = jnp.mean((t - mu) * (t - mu), axis=0, keepdims=True)
    return g * (t - mu) / jnp.sqrt(var + 1e-5) + be


def _layer_body(h_ref, p_ref, w1_ref, b1_ref, w2_ref, b2_ref, g_ref, be_ref,
                o_ref):
    o_ref[...] = _mlp_bn(h_ref[...], p_ref[0], p_ref[1], w1_ref[...],
                         b1_ref[...], w2_ref[...], b2_ref[...], g_ref[...],
                         be_ref[...])


def _final_body(h_ref, p_ref, w1_ref, b1_ref, w2_ref, b2_ref, g_ref, be_ref,
                batch_ref, fc1w_ref, fc1b_ref, fc2w_ref, fc2b_ref, o_ref):
    h = _mlp_bn(h_ref[...], p_ref[0], p_ref[1], w1_ref[...], b1_ref[...],
                w2_ref[...], b2_ref[...], g_ref[...], be_ref[...])
    seg = batch_ref[...]
    onehot = (seg == lax.broadcasted_iota(jnp.int32, (_N, _G), 1)
              ).astype(jnp.float32)
    pooled = lax.dot_general(onehot, h, (((0,), (0,)), ((), ())),
                             preferred_element_type=jnp.float32,
                             precision=jax.lax.Precision.HIGHEST)
    z = jnp.maximum(_dot(pooled, fc1w_ref[...]) + fc1b_ref[...], 0.0)
    logits = _dot(z, fc2w_ref[...]) + fc2b_ref[...]
    mx = jnp.max(logits, axis=-1, keepdims=True)
    s = logits - mx
    o_ref[...] = s - jnp.log(jnp.sum(jnp.exp(s), axis=-1, keepdims=True))


def _tc(body, out_shape, *args):
    return pl.pallas_call(
        body, out_shape=jax.ShapeDtypeStruct(out_shape, jnp.float32))(*args)


def kernel(x, edge_index, batch,
           W1_0, b1_0, W2_0, b2_0, gamma_0, beta_0,
           W1_1, b1_1, W2_1, b2_1, gamma_1, beta_1,
           W1_2, b1_2, W2_2, b2_2, gamma_2, beta_2,
           fc1_W, fc1_b, fc2_W, fc2_b):
    src = edge_index[0]
    dst = edge_index[1]
    zeros_f = jnp.zeros((_N, _F), jnp.float32)
    zeros_h = jnp.zeros((_N, _H), jnp.float32)
    batch2d = batch.reshape(_N, 1)
    r2 = lambda v: v.reshape(1, -1)

    p = _sc_agg(x, zeros_f, src, dst, _F)
    h = _tc(_layer_body, (_N, _H), x, p, W1_0, r2(b1_0), W2_0, r2(b2_0),
            r2(gamma_0), r2(beta_0))
    p = _sc_agg(h, zeros_h, src, dst, _H)
    h = _tc(_layer_body, (_N, _H), h, p, W1_1, r2(b1_1), W2_1, r2(b2_1),
            r2(gamma_1), r2(beta_1))
    p = _sc_agg(h, zeros_h, src, dst, _H)
    return _tc(_final_body, (_G, _C), h, p, W1_2, r2(b1_2), W2_2, r2(b2_2),
               r2(gamma_2), r2(beta_2), batch2d, fc1_W, r2(fc1_b),
               fc2_W, r2(fc2_b))
